# baseline (speedup 1.0000x reference)
import functools
import random as _pyrandom

import jax
import jax.numpy as jnp
import numpy as np
from jax import lax
from jax.experimental import pallas as pl
from jax.experimental.pallas import tpu as pltpu
from jax.experimental.pallas import tpu_sc as plsc

N_NODES = 10000
N_EDGES = 320000
N_PAD = 10240
RB = 640
NC_SC = 2
NS_SC = 16
NTILES = NC_SC * NS_SC
EG = 128
EPT = N_EDGES // NTILES
GPT = (EPT + EG - 1) // EG
EPT_PAD = GPT * EG
G0 = 96
G1 = 64
TG = NS_SC * (G0 + G1)
E_SEG_PAD = TG * EG
WB = 20
WPAD = WB * 128 * 128

_mt0 = _pyrandom.Random(0).getstate()[1]
MT_STATE0 = np.array(_mt0[:624], dtype=np.uint32)
MT_POS0 = int(_mt0[624])

_U = jnp.uint32


def _walk_body(mt_ref, src_ref, dst_ref, idx_ref, val_ref, mt_s):
    def _cp(i, c):
        mt_s[i] = mt_ref[i]
        return c
    lax.fori_loop(0, 624, _cp, 0)

    src = src_ref[...]
    dst = dst_ref[...]
    a1 = lax.broadcasted_iota(jnp.int32, (WB, 1), 0)
    a2 = lax.broadcasted_iota(jnp.int32, (WB, 128), 0)
    l2 = lax.broadcasted_iota(jnp.int32, (WB, 128), 1)
    a3 = lax.broadcasted_iota(jnp.int32, (WB, 128, 128), 0)
    b2 = lax.broadcasted_iota(jnp.int32, (128, 128), 0)
    lane1 = lax.broadcasted_iota(jnp.int32, (1, 128), 1)

    def _twist():
        upper = _U(0x80000000)
        lower = _U(0x7FFFFFFF)
        mata = _U(0x9908B0DF)

        def tb(kk, c):
            k1 = jnp.where(kk + 1 >= 624, kk - 623, kk + 1)
            k397 = jnp.where(kk + 397 >= 624, kk - 227, kk + 397)
            y = (mt_s[kk] & upper) | (mt_s[k1] & lower)
            v = mt_s[k397] ^ (y >> _U(1)) ^ ((y & _U(1)) * mata)
            mt_s[kk] = v
            return c
        lax.fori_loop(0, 624, tb, 0)

    def _step(cur, alive, pos):
        m3 = (src == cur).astype(jnp.int32)
        cnt2 = jnp.sum(m3, axis=2)
        blockcnt = jnp.sum(cnt2, axis=1, keepdims=True)
        deg = jnp.sum(blockcnt)
        go = jnp.logical_and(alive, deg > 0)
        bcs = blockcnt
        for sh in (1, 2, 4, 8, 16):
            bcs = bcs + jnp.where(a1 >= sh, pltpu.roll(bcs, sh, axis=0), 0)
        lcs = cnt2
        for sh in (1, 2, 4, 8, 16, 32, 64):
            lcs = lcs + jnp.where(l2 >= sh, pltpu.roll(lcs, sh, axis=1), 0)

        nd = jnp.maximum(deg, 1)
        k = jnp.int32(0)
        for b in range(32):
            k = jnp.where((nd >> b) != 0, jnp.int32(b + 1), k)
        shift = (jnp.int32(32) - k).astype(_U)
        nn = nd.astype(_U)

        def rb(t, carry):
            r, p = carry
            active = jnp.logical_and(go, r >= nn)
            ntw = jnp.logical_and(active, p >= 624)
            pl.when(ntw)(_twist)
            p2 = jnp.where(ntw, 0, p)
            y = mt_s[p2]
            y = y ^ (y >> _U(11))
            y = y ^ ((y << _U(7)) & _U(0x9D2C5680))
            y = y ^ ((y << _U(15)) & _U(0xEFC60000))
            y = y ^ (y >> _U(18))
            rn = y >> shift
            return (jnp.where(active, rn, r), jnp.where(active, p2 + 1, p))

        r, pos = lax.fori_loop(0, 48, rb, (nn, pos))
        t = r.astype(jnp.int32) + 1
        a = jnp.sum((bcs < t).astype(jnp.int32))
        prev_a = jnp.sum(jnp.where(a1 == a - 1, bcs, 0))
        t2 = t - prev_a
        lcs_a = jnp.sum(jnp.where(a2 == a, lcs, 0), axis=0, keepdims=True)
        b = jnp.sum((lcs_a < t2).astype(jnp.int32))
        prev_b = jnp.sum(jnp.where(lane1 == b - 1, lcs_a, 0))
        t3 = t2 - prev_b
        tmpm = jnp.sum(jnp.where(a3 == a, m3, 0), axis=0)
        rowm = jnp.sum(jnp.where(b2 == b, tmpm, 0), axis=0, keepdims=True)
        tmpd = jnp.sum(jnp.where(a3 == a, dst, 0), axis=0)
        dstrow = jnp.sum(jnp.where(b2 == b, tmpd, 0), axis=0, keepdims=True)
        lrow = rowm
        for sh in (1, 2, 4, 8, 16, 32, 64):
            lrow = lrow + jnp.where(lane1 >= sh, pltpu.roll(lrow, sh, axis=1), 0)
        c = jnp.sum((lrow < t3).astype(jnp.int32))
        nxt = jnp.sum(jnp.where(lane1 == c, dstrow, 0))
        cur = jnp.where(go, nxt, cur)
        return cur, go, pos

    def path_body(p, pos):
        idx_ref[p * 5] = jnp.int32(1)
        val_ref[p * 5] = jnp.int32(1)

        def sb(s, carry):
            cur, alive, pos = carry
            cur, go, pos = _step(cur, alive, pos)
            idx_ref[p * 5 + 1 + s] = jnp.where(go, cur, 0)
            val_ref[p * 5 + 1 + s] = go.astype(jnp.int32)
            return (cur, go, pos)

        cur, alive, pos = lax.fori_loop(
            0, 4, sb, (jnp.int32(1), jnp.bool_(True), pos))
        return pos

    lax.fori_loop(0, 10, path_body, jnp.int32(MT_POS0))


def _walk(mt0, srcw, dstw, interpret=False):
    return pl.pallas_call(
        _walk_body,
        in_specs=[
            pl.BlockSpec(memory_space=pltpu.SMEM),
            pl.BlockSpec(memory_space=pltpu.VMEM),
            pl.BlockSpec(memory_space=pltpu.VMEM),
        ],
        out_specs=[
            pl.BlockSpec(memory_space=pltpu.SMEM),
            pl.BlockSpec(memory_space=pltpu.SMEM),
        ],
        out_shape=[
            jax.ShapeDtypeStruct((50,), jnp.int32),
            jax.ShapeDtypeStruct((50,), jnp.int32),
        ],
        scratch_shapes=[pltpu.SMEM((624,), jnp.uint32)],
        interpret=interpret,
    )(mt0, srcw, dstw)


def _dinv_body(degp_ref, out_ref):
    ones = jnp.ones((NTILES, 1), jnp.float32)
    deg = lax.dot_general(degp_ref[...], ones, (((0,), (0,)), ((), ())),
                          preferred_element_type=jnp.float32)
    out_ref[...] = 1.0 / jnp.sqrt(deg + 1.0)


def _dinv(degp, interpret=False):
    return pl.pallas_call(
        _dinv_body,
        grid=(N_PAD // RB,),
        in_specs=[pl.BlockSpec((NTILES, RB), lambda r: (0, r))],
        out_specs=pl.BlockSpec((RB, 1), lambda r: (r, 0)),
        out_shape=jax.ShapeDtypeStruct((N_PAD, 1), jnp.float32),
        interpret=interpret,
    )(degp)


def _g1_body(cps, cw, x_ref, w_ref, dinv_ref, g3_ref):
    hw = jnp.dot(x_ref[...], w_ref[...], preferred_element_type=jnp.float32)
    dinv = dinv_ref[...]
    for t in range(cps):
        g3_ref[t] = dinv * hw[:, t * cw:(t + 1) * cw]


def _g1(xp, w, dinv, nc, cw, interpret=False):
    cps = 2 if (cw * nc) % 128 == 0 and nc > 1 else 1
    return pl.pallas_call(
        functools.partial(_g1_body, cps, cw),
        grid=(N_PAD // RB, nc // cps),
        in_specs=[
            pl.BlockSpec((RB, 128), lambda r, c: (r, 0)),
            pl.BlockSpec((128, cps * cw), lambda r, c: (0, c)),
            pl.BlockSpec((RB, 1), lambda r, c: (r, 0)),
        ],
        out_specs=pl.BlockSpec((cps, RB, cw), lambda r, c: (c, r, 0)),
        out_shape=jax.ShapeDtypeStruct((nc, N_PAD, cw), jnp.float32),
        interpret=interpret,
    )(xp, w, dinv)


def _glayer_body(ncp, cps, cw, segp_ref, g3p_ref, b_ref, dinv_ref, w_ref,
                 g3_ref):
    pre = jnp.concatenate(
        [segp_ref[t, 0] + segp_ref[t, 1] + g3p_ref[t] for t in range(ncp)],
        axis=1)
    dinv = dinv_ref[...]
    h = jnp.maximum(dinv * pre + b_ref[...], 0.0)
    hw = jnp.dot(h, w_ref[...], preferred_element_type=jnp.float32)
    for t in range(cps):
        g3_ref[t] = dinv * hw[:, t * cw:(t + 1) * cw]


def _glayer(segp, g3p, b, dinv, w, ncp, cwp, nc, cw, interpret=False):
    dpp = ncp * cwp
    cps = 2 if (cw * nc) % 128 == 0 and nc > 1 else 1
    return pl.pallas_call(
        functools.partial(_glayer_body, ncp, cps, cw),
        grid=(N_PAD // RB, nc // cps),
        in_specs=[
            pl.BlockSpec((ncp, 2, RB, cwp), lambda r, c: (0, 0, r, 0)),
            pl.BlockSpec((ncp, RB, cwp), lambda r, c: (0, r, 0)),
            pl.BlockSpec((1, dpp), lambda r, c: (0, 0)),
            pl.BlockSpec((RB, 1), lambda r, c: (r, 0)),
            pl.BlockSpec((dpp, cps * cw), lambda r, c: (0, c)),
        ],
        out_specs=pl.BlockSpec((cps, RB, cw), lambda r, c: (c, r, 0)),
        out_shape=jax.ShapeDtypeStruct((nc, N_PAD, cw), jnp.float32),
        interpret=interpret,
    )(segp, g3p, b, dinv, w)


def _agg_body(nc6, segp_ref, g3_ref, b_ref, dinv_ref, idx_ref, val_ref,
              out_ref, h_s):
    dinv = dinv_ref[...]
    pre = jnp.concatenate(
        [segp_ref[t, 0] + segp_ref[t, 1] + g3_ref[t] for t in range(nc6)],
        axis=1)
    h = jnp.maximum(dinv * pre + b_ref[...], 0.0)
    h_s[...] = h

    def body(i, carry):
        acc, ws = carry
        w = val_ref[i].astype(jnp.float32)
        row = h_s[pl.ds(idx_ref[i], 1), :]
        return (acc + w * row, ws + w)

    acc, ws = lax.fori_loop(
        0, 50, body, (jnp.zeros((1, 128), jnp.float32), jnp.float32(0.0)))
    agg = acc / ws
    out_ref[...] = agg / jnp.sqrt(jnp.sum(agg * agg))


def _agg(segp, g3, b, dinv, idxw, validw, interpret=False):
    return pl.pallas_call(
        functools.partial(_agg_body, NCS[5]),
        in_specs=[
            pl.BlockSpec(memory_space=pltpu.VMEM),
            pl.BlockSpec(memory_space=pltpu.VMEM),
            pl.BlockSpec(memory_space=pltpu.VMEM),
            pl.BlockSpec(memory_space=pltpu.VMEM),
            pl.BlockSpec(memory_space=pltpu.SMEM),
            pl.BlockSpec(memory_space=pltpu.SMEM),
        ],
        out_specs=pl.BlockSpec(memory_space=pltpu.VMEM),
        out_shape=jax.ShapeDtypeStruct((1, 128), jnp.float32),
        scratch_shapes=[pltpu.VMEM((N_PAD, 128), jnp.float32)],
        interpret=interpret,
    )(segp, g3, b, dinv, idxw, validw)


def _sc_mesh():
    return plsc.VectorSubcoreMesh(core_axis_name="c", subcore_axis_name="s",
                                  num_cores=NC_SC, num_subcores=NS_SC)


@functools.cache
def _make_deg():
    @functools.partial(
        pl.kernel,
        out_type=jax.ShapeDtypeStruct((NTILES, N_PAD), jnp.float32),
        mesh=_sc_mesh(),
        scratch_types=[
            pltpu.VMEM((GPT, EG), jnp.int32),
            pltpu.VMEM((N_PAD,), jnp.float32),
        ],
        compiler_params=pltpu.CompilerParams(needs_layout_passes=False),
    )
    def _deg_kernel(dst3, out, dst_v, hist):
        cid = lax.axis_index("c")
        sid = lax.axis_index("s")
        wid = cid * NS_SC + sid
        pltpu.sync_copy(dst3.at[wid], dst_v)
        z16 = jnp.zeros((16,), jnp.float32)

        def zh(i, c):
            hist[pl.ds(i * 16, 16)] = z16
            return c
        lax.fori_loop(0, N_PAD // 16, zh, 0)
        ones16 = jnp.ones((16,), jnp.float32)

        def hb(i, c):
            j = i // 8
            k = i % 8
            idx = dst_v[j, pl.ds(k * 16, 16)]
            plsc.addupdate_scatter(hist, [idx], ones16)
            return c
        lax.fori_loop(0, GPT * 8, hb, 0)
        pltpu.sync_copy(hist, out.at[wid])

    return _deg_kernel


@functools.cache
def _make_seg(nc, cw):
    @functools.partial(
        pl.kernel,
        out_type=jax.ShapeDtypeStruct((nc, 2, N_PAD, cw), jnp.float32),
        mesh=_sc_mesh(),
        scratch_types=[
            pltpu.VMEM((G0, EG), jnp.int32),
            pltpu.VMEM((G0, EG), jnp.int32),
            pltpu.VMEM((EG, cw), jnp.float32),
            pltpu.SemaphoreType.DMA,
            pltpu.VMEM_SHARED((N_PAD, cw), jnp.float32),
        ],
        compiler_params=pltpu.CompilerParams(
            use_tc_tiling_on_sc=(None if cw == 128 else False)),
    )
    def seg_kernel(g3, src3, dst3, zsrc, out, src_v, dst_v, rows, sem, acc):
        cid = lax.axis_index("c")
        sid = lax.axis_index("s")

        def emit(G, gstart):
            pltpu.sync_copy(src3.at[pl.ds(gstart, G)], src_v.at[pl.ds(0, G)])
            pltpu.sync_copy(dst3.at[pl.ds(gstart, G)], dst_v.at[pl.ds(0, G)])
            for c in range(nc):
                pltpu.sync_copy(zsrc, acc.at[pl.ds(sid * 640, 640)])
                plsc.subcore_barrier()

                tbl = g3.at[c]

                def grp(j, carry):
                    pltpu.async_copy(tbl.at[src_v.at[j]], rows, sem).wait()
                    pltpu.sync_copy(rows, acc.at[dst_v.at[j]], add=True)
                    return carry
                lax.fori_loop(0, G, grp, 0)

                plsc.subcore_barrier()
                pltpu.sync_copy(
                    acc.at[pl.ds(sid * 640, 640)],
                    out.at[c, cid, pl.ds(sid * 640, 640)])
                plsc.subcore_barrier()

        @pl.when(cid == 0)
        def _():
            emit(G0, sid * G0)

        @pl.when(cid == 1)
        def _():
            emit(G1, NS_SC * G0 + sid * G1)

    return seg_kernel


DOUTS = [512, 256, 128, 64, 32, 128]
NCS = [4, 2, 1, 1, 1, 1]
CWS = [128, 128, 128, 64, 32, 128]


def kernel(x, edge_index, edge_attr,
           W1, b1, W2, b2, W3, b3, W4, b4, W5, b5, W6, b6):
    del edge_attr
    f32 = jnp.float32
    src = edge_index[0].astype(jnp.int32)
    dst = edge_index[1].astype(jnp.int32)

    srcw = jnp.concatenate(
        [src, jnp.full((WPAD - N_EDGES,), -1, jnp.int32)]).reshape(WB, 128, 128)
    dstw = jnp.concatenate(
        [dst, jnp.zeros((WPAD - N_EDGES,), jnp.int32)]).reshape(WB, 128, 128)
    idxw, validw = _walk(jnp.asarray(MT_STATE0), srcw, dstw)

    npad_e = NTILES * EPT_PAD - N_EDGES
    dst3d = jnp.concatenate(
        [dst, jnp.full((npad_e,), N_PAD - 1, jnp.int32)]).reshape(
            NTILES, GPT, EG)
    nseg_pad = E_SEG_PAD - N_EDGES
    src3 = jnp.concatenate(
        [src, jnp.full((nseg_pad,), N_PAD - 1, jnp.int32)]).reshape(TG, EG)
    dst3 = jnp.concatenate(
        [dst, jnp.full((nseg_pad,), N_PAD - 1, jnp.int32)]).reshape(TG, EG)

    degp = _make_deg()(dst3d)
    dinv = _dinv(degp)

    xp = jnp.concatenate([x.astype(f32),
                          jnp.zeros((N_PAD - N_NODES, 128), f32)], axis=0)

    Ws = [W1, W2, W3, W4, W5, W6]
    bs = [b1, b2, b3, b4, b5, b6]
    Wp = [w.astype(f32) for w in Ws]
    bp = [b.astype(f32)[None, :] for b in bs]

    g = _g1(xp, Wp[0], dinv, NCS[0], CWS[0])
    for i in range(1, 6):
        zs = jnp.zeros((640, CWS[i - 1]), f32)
        seg = _make_seg(NCS[i - 1], CWS[i - 1])(g, src3, dst3, zs)
        g = _glayer(seg, g, bp[i - 1], dinv, Wp[i],
                    NCS[i - 1], CWS[i - 1], NCS[i], CWS[i])
    seg6 = _make_seg(NCS[5], CWS[5])(g, src3, dst3,
                                     jnp.zeros((640, CWS[5]), f32))
    return _agg(seg6, g, bp[5], dinv, idxw, validw)


# speedup vs baseline: 1.9023x; 1.9023x over previous
import functools
import random as _pyrandom

import jax
import jax.numpy as jnp
import numpy as np
from jax import lax
from jax.experimental import pallas as pl
from jax.experimental.pallas import tpu as pltpu
from jax.experimental.pallas import tpu_sc as plsc

N_NODES = 10000
N_EDGES = 320000
N_PAD = 10240
RB = 640
NC_SC = 2
NS_SC = 16
NTILES = NC_SC * NS_SC
EG = 128
EPT = N_EDGES // NTILES
GPT = (EPT + EG - 1) // EG
EPT_PAD = GPT * EG
G0 = 91
G1 = 66
TG = NS_SC * (G0 + G1)
E_SEG_PAD = TG * EG
WB = 20
WPAD = WB * 128 * 128

_mt0 = _pyrandom.Random(0).getstate()[1]
MT_STATE0 = np.array(_mt0[:624], dtype=np.uint32)
MT_POS0 = int(_mt0[624])

_U = jnp.uint32


def _walk_body(mt_ref, src_ref, dst_ref, idx_ref, val_ref, mt_s):
    def _cp(i, c):
        mt_s[i] = mt_ref[i]
        return c
    lax.fori_loop(0, 624, _cp, 0)

    src = src_ref[...]
    dst = dst_ref[...]
    a1 = lax.broadcasted_iota(jnp.int32, (WB, 1), 0)
    a2 = lax.broadcasted_iota(jnp.int32, (WB, 128), 0)
    l2 = lax.broadcasted_iota(jnp.int32, (WB, 128), 1)
    a3 = lax.broadcasted_iota(jnp.int32, (WB, 128, 128), 0)
    b2 = lax.broadcasted_iota(jnp.int32, (128, 128), 0)
    lane1 = lax.broadcasted_iota(jnp.int32, (1, 128), 1)

    def _twist():
        upper = _U(0x80000000)
        lower = _U(0x7FFFFFFF)
        mata = _U(0x9908B0DF)

        def tb(kk, c):
            k1 = jnp.where(kk + 1 >= 624, kk - 623, kk + 1)
            k397 = jnp.where(kk + 397 >= 624, kk - 227, kk + 397)
            y = (mt_s[kk] & upper) | (mt_s[k1] & lower)
            v = mt_s[k397] ^ (y >> _U(1)) ^ ((y & _U(1)) * mata)
            mt_s[kk] = v
            return c
        lax.fori_loop(0, 624, tb, 0)

    def _step(cur, alive, pos):
        m3 = (src == cur).astype(jnp.int32)
        cnt2 = jnp.sum(m3, axis=2)
        blockcnt = jnp.sum(cnt2, axis=1, keepdims=True)
        deg = jnp.sum(blockcnt)
        go = jnp.logical_and(alive, deg > 0)
        bcs = blockcnt
        for sh in (1, 2, 4, 8, 16):
            bcs = bcs + jnp.where(a1 >= sh, pltpu.roll(bcs, sh, axis=0), 0)
        lcs = cnt2
        for sh in (1, 2, 4, 8, 16, 32, 64):
            lcs = lcs + jnp.where(l2 >= sh, pltpu.roll(lcs, sh, axis=1), 0)

        nd = jnp.maximum(deg, 1)
        k = jnp.int32(0)
        for b in range(32):
            k = jnp.where((nd >> b) != 0, jnp.int32(b + 1), k)
        shift = (jnp.int32(32) - k).astype(_U)
        nn = nd.astype(_U)

        def rb(t, carry):
            r, p = carry
            active = jnp.logical_and(go, r >= nn)
            ntw = jnp.logical_and(active, p >= 624)
            pl.when(ntw)(_twist)
            p2 = jnp.where(ntw, 0, p)
            y = mt_s[p2]
            y = y ^ (y >> _U(11))
            y = y ^ ((y << _U(7)) & _U(0x9D2C5680))
            y = y ^ ((y << _U(15)) & _U(0xEFC60000))
            y = y ^ (y >> _U(18))
            rn = y >> shift
            return (jnp.where(active, rn, r), jnp.where(active, p2 + 1, p))

        r, pos = lax.fori_loop(0, 48, rb, (nn, pos))
        t = r.astype(jnp.int32) + 1
        a = jnp.sum((bcs < t).astype(jnp.int32))
        prev_a = jnp.sum(jnp.where(a1 == a - 1, bcs, 0))
        t2 = t - prev_a
        lcs_a = jnp.sum(jnp.where(a2 == a, lcs, 0), axis=0, keepdims=True)
        b = jnp.sum((lcs_a < t2).astype(jnp.int32))
        prev_b = jnp.sum(jnp.where(lane1 == b - 1, lcs_a, 0))
        t3 = t2 - prev_b
        tmpm = jnp.sum(jnp.where(a3 == a, m3, 0), axis=0)
        rowm = jnp.sum(jnp.where(b2 == b, tmpm, 0), axis=0, keepdims=True)
        tmpd = jnp.sum(jnp.where(a3 == a, dst, 0), axis=0)
        dstrow = jnp.sum(jnp.where(b2 == b, tmpd, 0), axis=0, keepdims=True)
        lrow = rowm
        for sh in (1, 2, 4, 8, 16, 32, 64):
            lrow = lrow + jnp.where(lane1 >= sh, pltpu.roll(lrow, sh, axis=1), 0)
        c = jnp.sum((lrow < t3).astype(jnp.int32))
        nxt = jnp.sum(jnp.where(lane1 == c, dstrow, 0))
        cur = jnp.where(go, nxt, cur)
        return cur, go, pos

    def path_body(p, pos):
        idx_ref[p * 5] = jnp.int32(1)
        val_ref[p * 5] = jnp.int32(1)

        def sb(s, carry):
            cur, alive, pos = carry
            cur, go, pos = _step(cur, alive, pos)
            idx_ref[p * 5 + 1 + s] = jnp.where(go, cur, 0)
            val_ref[p * 5 + 1 + s] = go.astype(jnp.int32)
            return (cur, go, pos)

        cur, alive, pos = lax.fori_loop(
            0, 4, sb, (jnp.int32(1), jnp.bool_(True), pos))
        return pos

    lax.fori_loop(0, 10, path_body, jnp.int32(MT_POS0))


def _walk(mt0, srcw, dstw, interpret=False):
    return pl.pallas_call(
        _walk_body,
        in_specs=[
            pl.BlockSpec(memory_space=pltpu.SMEM),
            pl.BlockSpec(memory_space=pltpu.VMEM),
            pl.BlockSpec(memory_space=pltpu.VMEM),
        ],
        out_specs=[
            pl.BlockSpec(memory_space=pltpu.SMEM),
            pl.BlockSpec(memory_space=pltpu.SMEM),
        ],
        out_shape=[
            jax.ShapeDtypeStruct((50,), jnp.int32),
            jax.ShapeDtypeStruct((50,), jnp.int32),
        ],
        scratch_shapes=[pltpu.SMEM((624,), jnp.uint32)],
        interpret=interpret,
    )(mt0, srcw, dstw)


def _dinv_body(degp_ref, out_ref):
    ones = jnp.ones((NTILES, 1), jnp.float32)
    deg = lax.dot_general(degp_ref[...], ones, (((0,), (0,)), ((), ())),
                          preferred_element_type=jnp.float32)
    out_ref[...] = 1.0 / jnp.sqrt(deg + 1.0)


def _dinv(degp, interpret=False):
    return pl.pallas_call(
        _dinv_body,
        grid=(N_PAD // RB,),
        in_specs=[pl.BlockSpec((NTILES, RB), lambda r: (0, r))],
        out_specs=pl.BlockSpec((RB, 1), lambda r: (r, 0)),
        out_shape=jax.ShapeDtypeStruct((N_PAD, 1), jnp.float32),
        interpret=interpret,
    )(degp)


def _g1_body(cps, cw, x_ref, w_ref, dinv_ref, g3_ref):
    hw = jnp.dot(x_ref[...], w_ref[...], preferred_element_type=jnp.float32)
    dinv = dinv_ref[...]
    for t in range(cps):
        g3_ref[t] = dinv * hw[:, t * cw:(t + 1) * cw]


def _g1(xp, w, dinv, nc, cw, interpret=False):
    cps = 2 if (cw * nc) % 128 == 0 and nc > 1 else 1
    return pl.pallas_call(
        functools.partial(_g1_body, cps, cw),
        grid=(N_PAD // RB, nc // cps),
        in_specs=[
            pl.BlockSpec((RB, 128), lambda r, c: (r, 0)),
            pl.BlockSpec((128, cps * cw), lambda r, c: (0, c)),
            pl.BlockSpec((RB, 1), lambda r, c: (r, 0)),
        ],
        out_specs=pl.BlockSpec((cps, RB, cw), lambda r, c: (c, r, 0)),
        out_shape=jax.ShapeDtypeStruct((nc, N_PAD, cw), jnp.float32),
        interpret=interpret,
    )(xp, w, dinv)


def _glayer_body(ncp, cps, cw, segp_ref, g3p_ref, b_ref, dinv_ref, w_ref,
                 g3_ref):
    pre = jnp.concatenate(
        [segp_ref[t, 0] + segp_ref[t, 1] + g3p_ref[t] for t in range(ncp)],
        axis=1)
    dinv = dinv_ref[...]
    h = jnp.maximum(dinv * pre + b_ref[...], 0.0)
    hw = jnp.dot(h, w_ref[...], preferred_element_type=jnp.float32)
    for t in range(cps):
        g3_ref[t] = dinv * hw[:, t * cw:(t + 1) * cw]


def _glayer(segp, g3p, b, dinv, w, ncp, cwp, nc, cw, interpret=False):
    dpp = ncp * cwp
    cps = 2 if (cw * nc) % 128 == 0 and nc > 1 else 1
    return pl.pallas_call(
        functools.partial(_glayer_body, ncp, cps, cw),
        grid=(N_PAD // RB, nc // cps),
        in_specs=[
            pl.BlockSpec((ncp, 2, RB, cwp), lambda r, c: (0, 0, r, 0)),
            pl.BlockSpec((ncp, RB, cwp), lambda r, c: (0, r, 0)),
            pl.BlockSpec((1, dpp), lambda r, c: (0, 0)),
            pl.BlockSpec((RB, 1), lambda r, c: (r, 0)),
            pl.BlockSpec((dpp, cps * cw), lambda r, c: (0, c)),
        ],
        out_specs=pl.BlockSpec((cps, RB, cw), lambda r, c: (c, r, 0)),
        out_shape=jax.ShapeDtypeStruct((nc, N_PAD, cw), jnp.float32),
        interpret=interpret,
    )(segp, g3p, b, dinv, w)


def _agg_body(nc6, segp_ref, g3_ref, b_ref, dinv_ref, idx_ref, val_ref,
              out_ref, h_s):
    dinv = dinv_ref[...]
    pre = jnp.concatenate(
        [segp_ref[t, 0] + segp_ref[t, 1] + g3_ref[t] for t in range(nc6)],
        axis=1)
    h = jnp.maximum(dinv * pre + b_ref[...], 0.0)
    h_s[...] = h

    def body(i, carry):
        acc, ws = carry
        w = val_ref[i].astype(jnp.float32)
        row = h_s[pl.ds(idx_ref[i], 1), :]
        return (acc + w * row, ws + w)

    acc, ws = lax.fori_loop(
        0, 50, body, (jnp.zeros((1, 128), jnp.float32), jnp.float32(0.0)))
    agg = acc / ws
    out_ref[...] = agg / jnp.sqrt(jnp.sum(agg * agg))


def _agg(segp, g3, b, dinv, idxw, validw, interpret=False):
    return pl.pallas_call(
        functools.partial(_agg_body, NCS[5]),
        in_specs=[
            pl.BlockSpec(memory_space=pltpu.VMEM),
            pl.BlockSpec(memory_space=pltpu.VMEM),
            pl.BlockSpec(memory_space=pltpu.VMEM),
            pl.BlockSpec(memory_space=pltpu.VMEM),
            pl.BlockSpec(memory_space=pltpu.SMEM),
            pl.BlockSpec(memory_space=pltpu.SMEM),
        ],
        out_specs=pl.BlockSpec(memory_space=pltpu.VMEM),
        out_shape=jax.ShapeDtypeStruct((1, 128), jnp.float32),
        scratch_shapes=[pltpu.VMEM((N_PAD, 128), jnp.float32)],
        interpret=interpret,
    )(segp, g3, b, dinv, idxw, validw)


def _sc_mesh():
    return plsc.VectorSubcoreMesh(core_axis_name="c", subcore_axis_name="s",
                                  num_cores=NC_SC, num_subcores=NS_SC)


@functools.cache
def _make_deg():
    @functools.partial(
        pl.kernel,
        out_type=jax.ShapeDtypeStruct((NTILES, N_PAD), jnp.float32),
        mesh=_sc_mesh(),
        scratch_types=[
            pltpu.VMEM((GPT, EG), jnp.int32),
            pltpu.VMEM((N_PAD,), jnp.float32),
        ],
        compiler_params=pltpu.CompilerParams(needs_layout_passes=False),
    )
    def _deg_kernel(dst3, out, dst_v, hist):
        cid = lax.axis_index("c")
        sid = lax.axis_index("s")
        wid = cid * NS_SC + sid
        pltpu.sync_copy(dst3.at[wid], dst_v)
        z16 = jnp.zeros((16,), jnp.float32)

        def zh(i, c):
            hist[pl.ds(i * 16, 16)] = z16
            return c
        lax.fori_loop(0, N_PAD // 16, zh, 0)
        ones16 = jnp.ones((16,), jnp.float32)

        def hb(i, c):
            j = i // 8
            k = i % 8
            idx = dst_v[j, pl.ds(k * 16, 16)]
            plsc.addupdate_scatter(hist, [idx], ones16)
            return c
        lax.fori_loop(0, GPT * 8, hb, 0)
        pltpu.sync_copy(hist, out.at[wid])

    return _deg_kernel


@functools.cache
def _make_seg(nc, cw):
    @functools.partial(
        pl.kernel,
        out_type=jax.ShapeDtypeStruct((nc, 2, N_PAD, cw), jnp.float32),
        mesh=_sc_mesh(),
        scratch_types=[
            pltpu.VMEM((G0, EG), jnp.int32),
            pltpu.VMEM((G0, EG), jnp.int32),
            pltpu.VMEM((EG, cw), jnp.float32),
            pltpu.VMEM((EG, cw), jnp.float32),
            pltpu.SemaphoreType.DMA,
            pltpu.SemaphoreType.DMA,
            pltpu.VMEM_SHARED((N_PAD, cw), jnp.float32),
        ],
        compiler_params=pltpu.CompilerParams(use_tc_tiling_on_sc=False),
    )
    def seg_kernel(g3, src3, dst3, zsrc, out, src_v, dst_v, rows0, rows1,
                   sem0, sem1, acc):
        cid = lax.axis_index("c")
        sid = lax.axis_index("s")

        def emit(G, gstart):
            pltpu.sync_copy(src3.at[pl.ds(gstart, G)], src_v.at[pl.ds(0, G)])
            pltpu.sync_copy(dst3.at[pl.ds(gstart, G)], dst_v.at[pl.ds(0, G)])
            nh = G // 2
            for c in range(nc):
                pltpu.sync_copy(zsrc, acc.at[pl.ds(sid * 640, 640)])
                plsc.subcore_barrier()

                tbl = g3.at[c]
                pltpu.async_copy(tbl.at[src_v.at[0]], rows0, sem0)

                def grp(t, carry):
                    j = 2 * t
                    pltpu.make_async_copy(
                        tbl.at[src_v.at[j]], rows0, sem0).wait()
                    pltpu.async_copy(tbl.at[src_v.at[j + 1]], rows1, sem1)
                    pltpu.sync_copy(rows0, acc.at[dst_v.at[j]], add=True)
                    pltpu.make_async_copy(
                        tbl.at[src_v.at[j + 1]], rows1, sem1).wait()
                    pltpu.async_copy(
                        tbl.at[src_v.at[jnp.minimum(j + 2, G - 1)]],
                        rows0, sem0)
                    pltpu.sync_copy(rows1, acc.at[dst_v.at[j + 1]], add=True)
                    return carry
                lax.fori_loop(0, nh, grp, 0)
                pltpu.make_async_copy(
                    tbl.at[src_v.at[G - 1]], rows0, sem0).wait()
                if G % 2 == 1:
                    pltpu.sync_copy(rows0, acc.at[dst_v.at[G - 1]], add=True)

                plsc.subcore_barrier()
                pltpu.sync_copy(
                    acc.at[pl.ds(sid * 640, 640)],
                    out.at[c, cid, pl.ds(sid * 640, 640)])
                plsc.subcore_barrier()

        @pl.when(cid == 0)
        def _():
            emit(G0, sid * G0)

        @pl.when(cid == 1)
        def _():
            emit(G1, NS_SC * G0 + sid * G1)

    return seg_kernel


DOUTS = [512, 256, 128, 64, 32, 128]
NCS = [8, 4, 2, 1, 1, 2]
CWS = [64, 64, 64, 64, 32, 64]


def kernel(x, edge_index, edge_attr,
           W1, b1, W2, b2, W3, b3, W4, b4, W5, b5, W6, b6):
    del edge_attr
    f32 = jnp.float32
    src = edge_index[0].astype(jnp.int32)
    dst = edge_index[1].astype(jnp.int32)

    srcw = jnp.concatenate(
        [src, jnp.full((WPAD - N_EDGES,), -1, jnp.int32)]).reshape(WB, 128, 128)
    dstw = jnp.concatenate(
        [dst, jnp.zeros((WPAD - N_EDGES,), jnp.int32)]).reshape(WB, 128, 128)
    idxw, validw = _walk(jnp.asarray(MT_STATE0), srcw, dstw)

    npad_e = NTILES * EPT_PAD - N_EDGES
    dst3d = jnp.concatenate(
        [dst, jnp.full((npad_e,), N_PAD - 1, jnp.int32)]).reshape(
            NTILES, GPT, EG)
    nseg_pad = E_SEG_PAD - N_EDGES
    src3 = jnp.concatenate(
        [src, jnp.full((nseg_pad,), N_PAD - 1, jnp.int32)]).reshape(TG, EG)
    dst3 = jnp.concatenate(
        [dst, jnp.full((nseg_pad,), N_PAD - 1, jnp.int32)]).reshape(TG, EG)

    degp = _make_deg()(dst3d)
    dinv = _dinv(degp)

    xp = jnp.concatenate([x.astype(f32),
                          jnp.zeros((N_PAD - N_NODES, 128), f32)], axis=0)

    Ws = [W1, W2, W3, W4, W5, W6]
    bs = [b1, b2, b3, b4, b5, b6]
    Wp = [w.astype(f32) for w in Ws]
    bp = [b.astype(f32)[None, :] for b in bs]

    g = _g1(xp, Wp[0], dinv, NCS[0], CWS[0])
    for i in range(1, 6):
        zs = jnp.zeros((640, CWS[i - 1]), f32)
        seg = _make_seg(NCS[i - 1], CWS[i - 1])(g, src3, dst3, zs)
        g = _glayer(seg, g, bp[i - 1], dinv, Wp[i],
                    NCS[i - 1], CWS[i - 1], NCS[i], CWS[i])
    seg6 = _make_seg(NCS[5], CWS[5])(g, src3, dst3,
                                     jnp.zeros((640, CWS[5]), f32))
    return _agg(seg6, g, bp[5], dinv, idxw, validw)
